# baseline (speedup 1.0000x reference)
import functools

import jax
import jax.numpy as jnp
from jax import lax
from jax.experimental import pallas as pl
from jax.experimental.pallas import tpu as pltpu
from jax.experimental.pallas import tpu_sc as plsc

_N = 100000
_NW = 32
_RPW = _N // _NW


@functools.partial(
    pl.kernel,
    out_type=jax.ShapeDtypeStruct((_N, 6, 8), jnp.float32),
    mesh=plsc.VectorSubcoreMesh(core_axis_name="c", subcore_axis_name="s"),
    compiler_params=pltpu.CompilerParams(use_tc_tiling_on_sc=True),
)
def _sc_copy(x_hbm, o_hbm):
    wid = lax.axis_index("s") * 2 + lax.axis_index("c")
    base = wid * _RPW
    pltpu.sync_copy(
        x_hbm.at[pl.ds(base, _RPW)],
        o_hbm.at[pl.ds(base, _RPW)],
    )


def _fix_body(x_ref, c_ref, o_ref):
    o_ref[...] = x_ref[...] + c_ref[...]


def kernel(x):
    t = jnp.arange(0, 336, 1, dtype=jnp.float32).reshape(7, 6, 8)
    addvals = jnp.concatenate(
        [t[jnp.array([0, 2, 1, 3, 4, 5, 6])], jnp.zeros((1, 6, 8), jnp.float32)], axis=0
    )
    y = _sc_copy(x)
    return pl.pallas_call(
        _fix_body,
        grid=(1,),
        in_specs=[
            pl.BlockSpec((8, 6, 8), lambda i: (0, 0, 0)),
            pl.BlockSpec((8, 6, 8), lambda i: (0, 0, 0)),
        ],
        out_specs=pl.BlockSpec((8, 6, 8), lambda i: (0, 0, 0)),
        out_shape=jax.ShapeDtypeStruct((_N, 6, 8), jnp.float32),
        input_output_aliases={0: 0},
    )(y, addvals)


# speedup vs baseline: 19.1978x; 19.1978x over previous
import jax
import jax.numpy as jnp
from jax.experimental import pallas as pl
from jax.experimental.pallas import tpu as pltpu

_N = 100000
_ROWS2D = 37500


def _fix_body(x_ref, c_ref, o_ref):
    o_ref[...] = x_ref[...] + c_ref[...]


def kernel(x):
    t = jnp.arange(0, 336, 1, dtype=jnp.float32).reshape(7, 6, 8)
    addflat = jnp.zeros((8 * 128,), jnp.float32)
    addflat = addflat.at[0:336].set(t[jnp.array([0, 2, 1, 3, 4, 5, 6])].reshape(336))
    addflat = addflat.reshape(8, 128)
    xr = x.reshape(_ROWS2D, 128)
    res = pl.pallas_call(
        _fix_body,
        grid=(1,),
        in_specs=[
            pl.BlockSpec((8, 128), lambda i: (0, 0)),
            pl.BlockSpec((8, 128), lambda i: (0, 0)),
        ],
        out_specs=pl.BlockSpec((8, 128), lambda i: (0, 0)),
        out_shape=jax.ShapeDtypeStruct((_ROWS2D, 128), jnp.float32),
        input_output_aliases={0: 0},
    )(xr, addflat)
    return res.reshape(_N, 6, 8)


# speedup vs baseline: 791.9379x; 41.2515x over previous
import jax
import jax.numpy as jnp
from jax.experimental import pallas as pl
from jax.experimental.pallas import tpu as pltpu

_N = 100000
_BL = 12800
_GRID = -(-_N // _BL)


def _body(x_ref, c_ref, o_ref):
    o_ref[...] = x_ref[...]
    @pl.when(pl.program_id(0) == 0)
    def _():
        o_ref[:, :, 0:128] = o_ref[:, :, 0:128] + c_ref[...]


def kernel(x):
    t = jnp.arange(0, 336, 1, dtype=jnp.float32).reshape(7, 6, 8)
    addvals = t[jnp.array([0, 2, 1, 3, 4, 5, 6])]
    caddT = jnp.zeros((6, 8, 128), jnp.float32).at[:, :, 0:7].set(
        addvals.transpose(1, 2, 0)
    )
    xt = jnp.transpose(x, (1, 2, 0))
    res = pl.pallas_call(
        _body,
        grid=(_GRID,),
        in_specs=[
            pl.BlockSpec((6, 8, _BL), lambda i: (0, 0, i)),
            pl.BlockSpec((6, 8, 128), lambda i: (0, 0, 0)),
        ],
        out_specs=pl.BlockSpec((6, 8, _BL), lambda i: (0, 0, i)),
        out_shape=jax.ShapeDtypeStruct((6, 8, _N), jnp.float32),
        compiler_params=pltpu.CompilerParams(
            dimension_semantics=("arbitrary",),
        ),
    )(xt, caddT)
    return jnp.transpose(res, (2, 0, 1))


# speedup vs baseline: 855.3427x; 1.0801x over previous
import jax
import jax.numpy as jnp
from jax.experimental import pallas as pl
from jax.experimental.pallas import tpu as pltpu

_N = 100000
_BL = 25600
_GRID = -(-_N // _BL)


def _body(x_ref, c_ref, o_ref):
    o_ref[...] = x_ref[...]
    @pl.when(pl.program_id(0) == 0)
    def _():
        o_ref[:, :, 0:128] = o_ref[:, :, 0:128] + c_ref[...]


def kernel(x):
    t = jnp.arange(0, 336, 1, dtype=jnp.float32).reshape(7, 6, 8)
    addvals = t[jnp.array([0, 2, 1, 3, 4, 5, 6])]
    caddT = jnp.zeros((6, 8, 128), jnp.float32).at[:, :, 0:7].set(
        addvals.transpose(1, 2, 0)
    )
    xt = jnp.transpose(x, (1, 2, 0))
    res = pl.pallas_call(
        _body,
        grid=(_GRID,),
        in_specs=[
            pl.BlockSpec((6, 8, _BL), lambda i: (0, 0, i)),
            pl.BlockSpec((6, 8, 128), lambda i: (0, 0, 0)),
        ],
        out_specs=pl.BlockSpec((6, 8, _BL), lambda i: (0, 0, i)),
        out_shape=jax.ShapeDtypeStruct((6, 8, _N), jnp.float32),
        compiler_params=pltpu.CompilerParams(
            dimension_semantics=("arbitrary",),
        ),
    )(xt, caddT)
    return jnp.transpose(res, (2, 0, 1))


# speedup vs baseline: 946.2546x; 1.1063x over previous
import jax
import jax.numpy as jnp
from jax.experimental import pallas as pl
from jax.experimental.pallas import tpu as pltpu

_N = 100000
_BL = 50048
_GRID = -(-_N // _BL)


def _body(x_ref, c_ref, o_ref):
    o_ref[...] = x_ref[...]
    @pl.when(pl.program_id(0) == 0)
    def _():
        o_ref[:, :, 0:128] = o_ref[:, :, 0:128] + c_ref[...]


def kernel(x):
    t = jnp.arange(0, 336, 1, dtype=jnp.float32).reshape(7, 6, 8)
    addvals = t[jnp.array([0, 2, 1, 3, 4, 5, 6])]
    caddT = jnp.zeros((6, 8, 128), jnp.float32).at[:, :, 0:7].set(
        addvals.transpose(1, 2, 0)
    )
    xt = jnp.transpose(x, (1, 2, 0))
    res = pl.pallas_call(
        _body,
        grid=(_GRID,),
        in_specs=[
            pl.BlockSpec((6, 8, _BL), lambda i: (0, 0, i)),
            pl.BlockSpec((6, 8, 128), lambda i: (0, 0, 0)),
        ],
        out_specs=pl.BlockSpec((6, 8, _BL), lambda i: (0, 0, i)),
        out_shape=jax.ShapeDtypeStruct((6, 8, _N), jnp.float32),
        compiler_params=pltpu.CompilerParams(
            dimension_semantics=("arbitrary",),
        ),
    )(xt, caddT)
    return jnp.transpose(res, (2, 0, 1))
